# baseline (speedup 1.0000x reference)
import functools

import jax
import jax.numpy as jnp
from jax import lax
from jax.experimental import pallas as pl
from jax.experimental.pallas import tpu as pltpu
from jax.experimental.pallas import tpu_sc as plsc

N_NODES = 10000
N_PAD = 10240
N_EDGES = 320000
D_FEAT = 128
D_EDGE = 16
H = 128

NC = 2
NS = 16
NW = NC * NS
E_PER_W = N_EDGES // NW
GC = 80
N_CH = E_PER_W // GC
ROWS_PER_TILE = N_PAD // NS

f32 = jnp.float32


@functools.lru_cache(maxsize=None)
def _sc_mesh():
    return plsc.VectorSubcoreMesh(core_axis_name="c", subcore_axis_name="s")


bf16 = jnp.bfloat16


def _gather_body(n_edges, gc, p_hbm, q_hbm, src_hbm, dst_hbm, outp_hbm,
                 outq_hbm,
                 idx_s0, idx_d0, idx_s1, idx_d1,
                 rows_p0, rows_q0, rows_p1, rows_q1,
                 sem_p0, sem_q0, sem_p1, sem_q1):
    e_per_w = n_edges // NW
    n_ch = e_per_w // gc
    GC = gc
    N_CH = n_ch
    wid = lax.axis_index("s") * NC + lax.axis_index("c")
    base0 = wid * e_per_w
    idx = ((idx_s0, idx_d0), (idx_s1, idx_d1))
    rows = ((rows_p0, rows_q0), (rows_p1, rows_q1))
    sems = ((sem_p0, sem_q0), (sem_p1, sem_q1))

    def fire(ci, b):
        base = base0 + ci * GC
        pltpu.sync_copy(src_hbm.at[pl.ds(base, GC)], idx[b][0])
        pltpu.sync_copy(dst_hbm.at[pl.ds(base, GC)], idx[b][1])
        pltpu.async_copy(p_hbm.at[idx[b][0]], rows[b][0], sems[b][0])
        pltpu.async_copy(q_hbm.at[idx[b][1]], rows[b][1], sems[b][1])

    def drain_store(ci, b):
        pltpu.make_async_copy(p_hbm.at[idx[b][0]], rows[b][0],
                              sems[b][0]).wait()
        pltpu.make_async_copy(q_hbm.at[idx[b][1]], rows[b][1],
                              sems[b][1]).wait()
        base = base0 + ci * GC
        pltpu.sync_copy(rows[b][0], outp_hbm.at[pl.ds(base, GC)])
        pltpu.sync_copy(rows[b][1], outq_hbm.at[pl.ds(base, GC)])

    fire(0, 0)

    def body(j, carry):
        c0 = j * 2
        fire(c0 + 1, 1)
        drain_store(c0, 0)

        @pl.when(c0 + 2 < N_CH)
        def _():
            fire(c0 + 2, 0)

        drain_store(c0 + 1, 1)
        return carry

    lax.fori_loop(0, N_CH // 2, body, 0)
    if N_CH % 2 == 1:
        drain_store(N_CH - 1, 0)


@functools.lru_cache(maxsize=None)
def _gather_pq_fn(n_edges=N_EDGES, gc=GC):
    return pl.kernel(
        functools.partial(_gather_body, n_edges, gc),
        out_type=(jax.ShapeDtypeStruct((n_edges, H), f32),
                  jax.ShapeDtypeStruct((n_edges, H), f32)),
        mesh=_sc_mesh(),
        scratch_types=[
            pltpu.VMEM((gc,), jnp.int32),
            pltpu.VMEM((gc,), jnp.int32),
            pltpu.VMEM((gc,), jnp.int32),
            pltpu.VMEM((gc,), jnp.int32),
            pltpu.VMEM((gc, H), f32),
            pltpu.VMEM((gc, H), f32),
            pltpu.VMEM((gc, H), f32),
            pltpu.VMEM((gc, H), f32),
            pltpu.SemaphoreType.DMA,
            pltpu.SemaphoreType.DMA,
            pltpu.SemaphoreType.DMA,
            pltpu.SemaphoreType.DMA,
        ],
    )


def _scatter_body(n_edges, gc, vals_hbm, dst_hbm, zeros_hbm, out_hbm,
                  idx0, idx1, val0, val1,
                  semi0, semv0, semi1, semv1, acc_sh):
    e_per_w = n_edges // NW
    GC = gc
    N_CH = e_per_w // gc
    cid = lax.axis_index("c")
    sid = lax.axis_index("s")
    wid = sid * NC + cid
    row0 = sid * ROWS_PER_TILE
    base0 = wid * e_per_w
    idx = (idx0, idx1)
    val = (val0, val1)
    semi = (semi0, semi1)
    semv = (semv0, semv1)

    pltpu.sync_copy(zeros_hbm.at[pl.ds(row0, ROWS_PER_TILE)],
                    acc_sh.at[pl.ds(row0, ROWS_PER_TILE)])
    plsc.subcore_barrier()

    def fire(ci, b):
        base = base0 + ci * GC
        pltpu.async_copy(dst_hbm.at[pl.ds(base, GC)], idx[b], semi[b])
        pltpu.async_copy(vals_hbm.at[pl.ds(base, GC)], val[b], semv[b])

    def scat(ci, b):
        base = base0 + ci * GC
        pltpu.make_async_copy(dst_hbm.at[pl.ds(base, GC)], idx[b],
                              semi[b]).wait()
        pltpu.make_async_copy(vals_hbm.at[pl.ds(base, GC)], val[b],
                              semv[b]).wait()
        pltpu.sync_copy(val[b], acc_sh.at[idx[b]], add=True)

    fire(0, 0)

    def body(j, carry):
        c0 = j * 2
        fire(c0 + 1, 1)
        scat(c0, 0)

        @pl.when(c0 + 2 < N_CH)
        def _():
            fire(c0 + 2, 0)

        scat(c0 + 1, 1)
        return carry

    lax.fori_loop(0, N_CH // 2, body, 0)
    if N_CH % 2 == 1:
        scat(N_CH - 1, 0)
    plsc.subcore_barrier()

    pltpu.sync_copy(acc_sh.at[pl.ds(row0, ROWS_PER_TILE)],
                    out_hbm.at[cid, pl.ds(row0, ROWS_PER_TILE)])


@functools.lru_cache(maxsize=None)
def _scatter_add_fn(n_edges=N_EDGES, gc=GC):
    return pl.kernel(
        functools.partial(_scatter_body, n_edges, gc),
        out_type=jax.ShapeDtypeStruct((NC, N_PAD, H), f32),
        mesh=_sc_mesh(),
        scratch_types=[
            pltpu.VMEM((gc,), jnp.int32),
            pltpu.VMEM((gc,), jnp.int32),
            pltpu.VMEM((gc, H), f32),
            pltpu.VMEM((gc, H), f32),
            pltpu.SemaphoreType.DMA,
            pltpu.SemaphoreType.DMA,
            pltpu.SemaphoreType.DMA,
            pltpu.SemaphoreType.DMA,
            pltpu.VMEM_SHARED((N_PAD, H), f32),
        ],
    )


def _pq_body(x_ref, wa_ref, wb_ref, p_ref, q_ref):
    xv = x_ref[...]
    p_ref[...] = jnp.dot(xv, wa_ref[...], preferred_element_type=f32)
    q_ref[...] = jnp.dot(xv, wb_ref[...], preferred_element_type=f32)


BN1 = 2000


def _compute_pq(x, wa, wb):
    return pl.pallas_call(
        _pq_body,
        grid=(N_NODES // BN1,),
        in_specs=[
            pl.BlockSpec((BN1, D_FEAT), lambda i: (i, 0)),
            pl.BlockSpec((D_FEAT, H), lambda i: (0, 0)),
            pl.BlockSpec((D_FEAT, H), lambda i: (0, 0)),
        ],
        out_specs=[
            pl.BlockSpec((BN1, H), lambda i: (i, 0)),
            pl.BlockSpec((BN1, H), lambda i: (i, 0)),
        ],
        out_shape=[
            jax.ShapeDtypeStruct((N_NODES, H), f32),
            jax.ShapeDtypeStruct((N_NODES, H), f32),
        ],
    )(x, wa, wb)


BE = 2560


def _edge_body(ps_ref, qd_ref, ea_ref, we1e_ref, be1_ref, we2_ref, be2_ref,
               wn1b_ref, out_ref, msg_ref):
    r = jnp.dot(ea_ref[...], we1e_ref[...], preferred_element_type=f32)
    h = jnp.maximum(ps_ref[...] + qd_ref[...] + r + be1_ref[...], 0.0)
    new_e = jnp.dot(h, we2_ref[...], preferred_element_type=f32) \
        + be2_ref[...]
    out_ref[...] = new_e
    msg_ref[...] = jnp.dot(new_e, wn1b_ref[...], preferred_element_type=f32)


def _edge_mlp(psrc, qdst, ea, we1e, be1, we2, be2, wn1b):
    n_edges = psrc.shape[0]
    return pl.pallas_call(
        _edge_body,
        grid=(n_edges // BE,),
        in_specs=[
            pl.BlockSpec((BE, H), lambda i: (i, 0)),
            pl.BlockSpec((BE, H), lambda i: (i, 0)),
            pl.BlockSpec((BE, D_EDGE), lambda i: (i, 0)),
            pl.BlockSpec((D_EDGE, H), lambda i: (0, 0)),
            pl.BlockSpec((1, H), lambda i: (0, 0)),
            pl.BlockSpec((H, D_EDGE), lambda i: (0, 0)),
            pl.BlockSpec((1, D_EDGE), lambda i: (0, 0)),
            pl.BlockSpec((D_EDGE, H), lambda i: (0, 0)),
        ],
        out_specs=[
            pl.BlockSpec((BE, D_EDGE), lambda i: (i, 0)),
            pl.BlockSpec((BE, H), lambda i: (i, 0)),
        ],
        out_shape=[
            jax.ShapeDtypeStruct((n_edges, D_EDGE), f32),
            jax.ShapeDtypeStruct((n_edges, H), f32),
        ],
    )(psrc, qdst, ea, we1e, be1, we2, be2, wn1b)


BN2 = 2000


def _node_body(x_ref, *refs):
    out_ref = refs[-1]
    wn1a_ref, bn1_ref, wn2_ref, bn2_ref = refs[-5:-1]
    agg = refs[0][0]
    for r in refs[1:-5]:
        agg = agg + r[0]
    hn = jnp.maximum(
        jnp.dot(x_ref[...], wn1a_ref[...], preferred_element_type=f32)
        + agg + bn1_ref[...], 0.0)
    out_ref[...] = jnp.dot(hn, wn2_ref[...], preferred_element_type=f32) \
        + bn2_ref[...]


def _node_mlp(x, agg_parts, wn1a, bn1, wn2, bn2):
    part_specs = []
    part_args = []
    for part in agg_parts:
        for c in range(NC):
            part_specs.append(
                pl.BlockSpec((1, BN2, H), lambda i, c=c: (c, i, 0)))
            part_args.append(part)
    return pl.pallas_call(
        _node_body,
        grid=(N_NODES // BN2,),
        in_specs=[
            pl.BlockSpec((BN2, D_FEAT), lambda i: (i, 0)),
            *part_specs,
            pl.BlockSpec((D_FEAT, H), lambda i: (0, 0)),
            pl.BlockSpec((1, H), lambda i: (0, 0)),
            pl.BlockSpec((H, D_FEAT), lambda i: (0, 0)),
            pl.BlockSpec((1, D_FEAT), lambda i: (0, 0)),
        ],
        out_specs=pl.BlockSpec((BN2, D_FEAT), lambda i: (i, 0)),
        out_shape=jax.ShapeDtypeStruct((N_NODES, D_FEAT), f32),
    )(x, *part_args, wn1a, bn1, wn2, bn2)


def kernel(x, edge_index, edge_attr, We1, be1, We2, be2, Wn1, bn1, Wn2, bn2):
    src = edge_index[0].astype(jnp.int32)
    dst = edge_index[1].astype(jnp.int32)
    we1a = We1[:D_FEAT]
    we1b = We1[D_FEAT:2 * D_FEAT]
    we1e = We1[2 * D_FEAT:]
    wn1a = Wn1[:D_FEAT]
    wn1b = Wn1[D_FEAT:]

    p, q = _compute_pq(x, we1a, we1b)
    zeros = jnp.zeros((N_PAD, H), f32)
    splits = [(0, 81920), (81920, 79360), (161280, 79360), (240640, 79360)]
    gc = 80
    parts = []
    new_es = []
    for off, eh in splits:
        src_s = src[off:off + eh]
        dst_s = dst[off:off + eh]
        ea_s = edge_attr[off:off + eh]
        psrc, qdst = _gather_pq_fn(eh, gc)(p, q, src_s, dst_s)
        new_e, msgs = _edge_mlp(psrc, qdst, ea_s, we1e,
                                be1.reshape(1, H), We2,
                                be2.reshape(1, D_EDGE), wn1b)
        parts.append(_scatter_add_fn(eh, gc)(msgs, dst_s, zeros))
        new_es.append(new_e)
    new_edge_attr = jnp.concatenate(new_es, axis=0)
    new_x = _node_mlp(x, parts, wn1a, bn1.reshape(1, H), Wn2,
                      bn2.reshape(1, D_FEAT))
    return (new_x, new_edge_attr)


# speedup vs baseline: 1.0436x; 1.0436x over previous
import functools

import jax
import jax.numpy as jnp
from jax import lax
from jax.experimental import pallas as pl
from jax.experimental.pallas import tpu as pltpu
from jax.experimental.pallas import tpu_sc as plsc

N_NODES = 10000
N_PAD = 10240
N_EDGES = 320000
D_FEAT = 128
D_EDGE = 16
H = 128

NC = 2
NS = 16
NW = NC * NS
E_PER_W = N_EDGES // NW
GC = 80
N_CH = E_PER_W // GC
ROWS_PER_TILE = N_PAD // NS

f32 = jnp.float32


@functools.lru_cache(maxsize=None)
def _sc_mesh():
    return plsc.VectorSubcoreMesh(core_axis_name="c", subcore_axis_name="s")


bf16 = jnp.bfloat16


def _gather_body(n_edges, gc, p_hbm, q_hbm, src_hbm, dst_hbm, outp_hbm,
                 outq_hbm,
                 idx_s0, idx_d0, idx_s1, idx_d1,
                 rows_p0, rows_q0, rows_p1, rows_q1,
                 sem_p0, sem_q0, sem_p1, sem_q1):
    e_per_w = n_edges // NW
    n_ch = e_per_w // gc
    GC = gc
    N_CH = n_ch
    wid = lax.axis_index("s") * NC + lax.axis_index("c")
    base0 = wid * e_per_w
    idx = ((idx_s0, idx_d0), (idx_s1, idx_d1))
    rows = ((rows_p0, rows_q0), (rows_p1, rows_q1))
    sems = ((sem_p0, sem_q0), (sem_p1, sem_q1))

    def fire(ci, b):
        base = base0 + ci * GC
        pltpu.sync_copy(src_hbm.at[pl.ds(base, GC)], idx[b][0])
        pltpu.sync_copy(dst_hbm.at[pl.ds(base, GC)], idx[b][1])
        pltpu.async_copy(p_hbm.at[idx[b][0]], rows[b][0], sems[b][0])
        pltpu.async_copy(q_hbm.at[idx[b][1]], rows[b][1], sems[b][1])

    def drain_store(ci, b):
        pltpu.make_async_copy(p_hbm.at[idx[b][0]], rows[b][0],
                              sems[b][0]).wait()
        pltpu.make_async_copy(q_hbm.at[idx[b][1]], rows[b][1],
                              sems[b][1]).wait()
        base = base0 + ci * GC
        pltpu.sync_copy(rows[b][0], outp_hbm.at[pl.ds(base, GC)])
        pltpu.sync_copy(rows[b][1], outq_hbm.at[pl.ds(base, GC)])

    fire(0, 0)

    def body(j, carry):
        c0 = j * 2
        fire(c0 + 1, 1)
        drain_store(c0, 0)

        @pl.when(c0 + 2 < N_CH)
        def _():
            fire(c0 + 2, 0)

        drain_store(c0 + 1, 1)
        return carry

    lax.fori_loop(0, N_CH // 2, body, 0)
    if N_CH % 2 == 1:
        drain_store(N_CH - 1, 0)


@functools.lru_cache(maxsize=None)
def _gather_pq_fn(n_edges=N_EDGES, gc=GC):
    return pl.kernel(
        functools.partial(_gather_body, n_edges, gc),
        out_type=(jax.ShapeDtypeStruct((n_edges, H), f32),
                  jax.ShapeDtypeStruct((n_edges, H), f32)),
        mesh=_sc_mesh(),
        scratch_types=[
            pltpu.VMEM((gc,), jnp.int32),
            pltpu.VMEM((gc,), jnp.int32),
            pltpu.VMEM((gc,), jnp.int32),
            pltpu.VMEM((gc,), jnp.int32),
            pltpu.VMEM((gc, H), f32),
            pltpu.VMEM((gc, H), f32),
            pltpu.VMEM((gc, H), f32),
            pltpu.VMEM((gc, H), f32),
            pltpu.SemaphoreType.DMA,
            pltpu.SemaphoreType.DMA,
            pltpu.SemaphoreType.DMA,
            pltpu.SemaphoreType.DMA,
        ],
    )


def _scatter_body(n_edges, gc, vals_hbm, dst_hbm, zeros_hbm, out_hbm,
                  idx0, idx1, val0, val1,
                  semi0, semv0, semi1, semv1, acc_sh):
    e_per_w = n_edges // NW
    GC = gc
    N_CH = e_per_w // gc
    cid = lax.axis_index("c")
    sid = lax.axis_index("s")
    wid = sid * NC + cid
    row0 = sid * ROWS_PER_TILE
    base0 = wid * e_per_w
    idx = (idx0, idx1)
    val = (val0, val1)
    semi = (semi0, semi1)
    semv = (semv0, semv1)

    pltpu.sync_copy(zeros_hbm.at[pl.ds(row0, ROWS_PER_TILE)],
                    acc_sh.at[pl.ds(row0, ROWS_PER_TILE)])
    plsc.subcore_barrier()

    def fire(ci, b):
        base = base0 + ci * GC
        pltpu.async_copy(dst_hbm.at[pl.ds(base, GC)], idx[b], semi[b])
        pltpu.async_copy(vals_hbm.at[pl.ds(base, GC)], val[b], semv[b])

    def scat(ci, b):
        base = base0 + ci * GC
        pltpu.make_async_copy(dst_hbm.at[pl.ds(base, GC)], idx[b],
                              semi[b]).wait()
        pltpu.make_async_copy(vals_hbm.at[pl.ds(base, GC)], val[b],
                              semv[b]).wait()
        pltpu.sync_copy(val[b], acc_sh.at[idx[b]], add=True)

    fire(0, 0)

    def body(j, carry):
        c0 = j * 2
        fire(c0 + 1, 1)
        scat(c0, 0)

        @pl.when(c0 + 2 < N_CH)
        def _():
            fire(c0 + 2, 0)

        scat(c0 + 1, 1)
        return carry

    lax.fori_loop(0, N_CH // 2, body, 0)
    if N_CH % 2 == 1:
        scat(N_CH - 1, 0)
    plsc.subcore_barrier()

    pltpu.sync_copy(acc_sh.at[pl.ds(row0, ROWS_PER_TILE)],
                    out_hbm.at[cid, pl.ds(row0, ROWS_PER_TILE)])


@functools.lru_cache(maxsize=None)
def _scatter_add_fn(n_edges=N_EDGES, gc=GC):
    return pl.kernel(
        functools.partial(_scatter_body, n_edges, gc),
        out_type=jax.ShapeDtypeStruct((NC, N_PAD, H), f32),
        mesh=_sc_mesh(),
        scratch_types=[
            pltpu.VMEM((gc,), jnp.int32),
            pltpu.VMEM((gc,), jnp.int32),
            pltpu.VMEM((gc, H), f32),
            pltpu.VMEM((gc, H), f32),
            pltpu.SemaphoreType.DMA,
            pltpu.SemaphoreType.DMA,
            pltpu.SemaphoreType.DMA,
            pltpu.SemaphoreType.DMA,
            pltpu.VMEM_SHARED((N_PAD, H), f32),
        ],
    )


def _pq_body(x_ref, wa_ref, wb_ref, p_ref, q_ref):
    xv = x_ref[...]
    p_ref[...] = jnp.dot(xv, wa_ref[...], preferred_element_type=f32)
    q_ref[...] = jnp.dot(xv, wb_ref[...], preferred_element_type=f32)


BN1 = 2000


def _compute_pq(x, wa, wb):
    return pl.pallas_call(
        _pq_body,
        grid=(N_NODES // BN1,),
        in_specs=[
            pl.BlockSpec((BN1, D_FEAT), lambda i: (i, 0)),
            pl.BlockSpec((D_FEAT, H), lambda i: (0, 0)),
            pl.BlockSpec((D_FEAT, H), lambda i: (0, 0)),
        ],
        out_specs=[
            pl.BlockSpec((BN1, H), lambda i: (i, 0)),
            pl.BlockSpec((BN1, H), lambda i: (i, 0)),
        ],
        out_shape=[
            jax.ShapeDtypeStruct((N_NODES, H), f32),
            jax.ShapeDtypeStruct((N_NODES, H), f32),
        ],
    )(x, wa, wb)


BE = 2560


def _edge_body(ps_ref, qd_ref, ea_ref, we1e_ref, be1_ref, we2_ref, be2_ref,
               wn1b_ref, out_ref, msg_ref):
    r = jnp.dot(ea_ref[...], we1e_ref[...], preferred_element_type=f32)
    h = jnp.maximum(ps_ref[...] + qd_ref[...] + r + be1_ref[...], 0.0)
    new_e = jnp.dot(h, we2_ref[...], preferred_element_type=f32) \
        + be2_ref[...]
    out_ref[...] = new_e
    msg_ref[...] = jnp.dot(new_e, wn1b_ref[...], preferred_element_type=f32)


def _edge_mlp(psrc, qdst, ea, we1e, be1, we2, be2, wn1b):
    n_edges = psrc.shape[0]
    return pl.pallas_call(
        _edge_body,
        grid=(n_edges // BE,),
        in_specs=[
            pl.BlockSpec((BE, H), lambda i: (i, 0)),
            pl.BlockSpec((BE, H), lambda i: (i, 0)),
            pl.BlockSpec((BE, D_EDGE), lambda i: (i, 0)),
            pl.BlockSpec((D_EDGE, H), lambda i: (0, 0)),
            pl.BlockSpec((1, H), lambda i: (0, 0)),
            pl.BlockSpec((H, D_EDGE), lambda i: (0, 0)),
            pl.BlockSpec((1, D_EDGE), lambda i: (0, 0)),
            pl.BlockSpec((D_EDGE, H), lambda i: (0, 0)),
        ],
        out_specs=[
            pl.BlockSpec((BE, D_EDGE), lambda i: (i, 0)),
            pl.BlockSpec((BE, H), lambda i: (i, 0)),
        ],
        out_shape=[
            jax.ShapeDtypeStruct((n_edges, D_EDGE), f32),
            jax.ShapeDtypeStruct((n_edges, H), f32),
        ],
    )(psrc, qdst, ea, we1e, be1, we2, be2, wn1b)


BN2 = 2000


def _node_body(x_ref, *refs):
    out_ref = refs[-1]
    wn1a_ref, bn1_ref, wn2_ref, bn2_ref = refs[-5:-1]
    agg = refs[0][0]
    for r in refs[1:-5]:
        agg = agg + r[0]
    hn = jnp.maximum(
        jnp.dot(x_ref[...], wn1a_ref[...], preferred_element_type=f32)
        + agg + bn1_ref[...], 0.0)
    out_ref[...] = jnp.dot(hn, wn2_ref[...], preferred_element_type=f32) \
        + bn2_ref[...]


def _node_mlp(x, agg_parts, wn1a, bn1, wn2, bn2):
    part_specs = []
    part_args = []
    for part in agg_parts:
        for c in range(NC):
            part_specs.append(
                pl.BlockSpec((1, BN2, H), lambda i, c=c: (c, i, 0)))
            part_args.append(part)
    return pl.pallas_call(
        _node_body,
        grid=(N_NODES // BN2,),
        in_specs=[
            pl.BlockSpec((BN2, D_FEAT), lambda i: (i, 0)),
            *part_specs,
            pl.BlockSpec((D_FEAT, H), lambda i: (0, 0)),
            pl.BlockSpec((1, H), lambda i: (0, 0)),
            pl.BlockSpec((H, D_FEAT), lambda i: (0, 0)),
            pl.BlockSpec((1, D_FEAT), lambda i: (0, 0)),
        ],
        out_specs=pl.BlockSpec((BN2, D_FEAT), lambda i: (i, 0)),
        out_shape=jax.ShapeDtypeStruct((N_NODES, D_FEAT), f32),
    )(x, *part_args, wn1a, bn1, wn2, bn2)


def kernel(x, edge_index, edge_attr, We1, be1, We2, be2, Wn1, bn1, Wn2, bn2):
    src = edge_index[0].astype(jnp.int32)
    dst = edge_index[1].astype(jnp.int32)
    we1a = We1[:D_FEAT]
    we1b = We1[D_FEAT:2 * D_FEAT]
    we1e = We1[2 * D_FEAT:]
    wn1a = Wn1[:D_FEAT]
    wn1b = Wn1[D_FEAT:]

    p, q = _compute_pq(x, we1a, we1b)
    zeros = jnp.zeros((N_PAD, H), f32)
    splits = [(0, 163840, 160), (163840, 156160, 80)]
    gc = 80
    parts = []
    new_es = []
    for off, eh, ggc in splits:
        src_s = src[off:off + eh]
        dst_s = dst[off:off + eh]
        ea_s = edge_attr[off:off + eh]
        psrc, qdst = _gather_pq_fn(eh, ggc)(p, q, src_s, dst_s)
        new_e, msgs = _edge_mlp(psrc, qdst, ea_s, we1e,
                                be1.reshape(1, H), We2,
                                be2.reshape(1, D_EDGE), wn1b)
        parts.append(_scatter_add_fn(eh, gc)(msgs, dst_s, zeros))
        new_es.append(new_e)
    new_edge_attr = jnp.concatenate(new_es, axis=0)
    new_x = _node_mlp(x, parts, wn1a, bn1.reshape(1, H), Wn2,
                      bn2.reshape(1, D_FEAT))
    return (new_x, new_edge_attr)
